# baseline (speedup 1.0000x reference)
import jax
import jax.numpy as jnp
from jax.experimental import pallas as pl
from jax.experimental.pallas import tpu as pltpu

_B, _VOC, _D, _K = 4096, 1000, 16, 20
_BB = 128


def _log_sigmoid(x):
    return jnp.minimum(x, 0.0) - jnp.log1p(jnp.exp(-jnp.abs(x)))


def _body(vi_ref, vo_ref, neg_ref, V_ref, U_ref, out_ref):
    V = V_ref[...]
    U = U_ref[...]
    vi_e = jnp.dot(vi_ref[...], V, preferred_element_type=jnp.float32)
    vo_e = jnp.dot(vo_ref[...], U, preferred_element_type=jnp.float32)
    left = _log_sigmoid(jnp.sum(vi_e * vo_e, axis=1))
    W = jax.lax.dot_general(vi_e, U, (((1,), (1,)), ((), ())),
                            preferred_element_type=jnp.float32)
    right = jnp.zeros((_BB,), jnp.float32)
    for k in range(_K):
        bm_k = jnp.sum(neg_ref[:, k, :] * W, axis=1)
        right = right + _log_sigmoid(-bm_k)
    partial = -jnp.sum(left + right) * (1.0 / _B)

    @pl.when(pl.program_id(0) == 0)
    def _():
        out_ref[0, 0] = 0.0

    out_ref[0, 0] += partial


def kernel(vi, vo, neg_samples, V, U):
    out = pl.pallas_call(
        _body,
        grid=(_B // _BB,),
        in_specs=[
            pl.BlockSpec((_BB, _VOC), lambda i: (i, 0)),
            pl.BlockSpec((_BB, _VOC), lambda i: (i, 0)),
            pl.BlockSpec((_BB, _K, _VOC), lambda i: (i, 0, 0)),
            pl.BlockSpec((_VOC, _D), lambda i: (0, 0)),
            pl.BlockSpec((_VOC, _D), lambda i: (0, 0)),
        ],
        out_specs=pl.BlockSpec(memory_space=pltpu.SMEM),
        out_shape=jax.ShapeDtypeStruct((1, 1), jnp.float32),
    )(vi, vo, neg_samples, V, U)
    return out[0, 0]
